# baseline (speedup 1.0000x reference)
import functools

import jax
import jax.numpy as jnp
from jax import lax
from jax.experimental import pallas as pl
from jax.experimental.pallas import tpu as pltpu


def _dblock_kernel(x_ref, w1_ref, w2_ref, w3_ref, o_ref, col_a, col_b,
                   *, C, H, W):
    f32 = jnp.float32
    bf16 = jnp.bfloat16
    HW = H * W

    idx = lax.broadcasted_iota(jnp.int32, (1, HW), 1)
    row = idx // W
    col = idx - row * W

    def maski(dr, dc):
        m = None
        if dr < 0:
            m = row >= -dr
        elif dr > 0:
            m = row < H - dr
        if dc < 0:
            mc = col >= -dc
        elif dc > 0:
            mc = col < W - dc
        else:
            mc = None
        if m is None:
            m = mc
        elif mc is not None:
            m = m & mc
        return jnp.where(m, jnp.int32(-1), jnp.int32(0))

    def build_col(col_ref, cur, d):
        cb = cur.astype(bf16)
        ci = pltpu.bitcast(cb, jnp.int32)
        for kh in range(3):
            dr = (kh - 1) * d
            for kw in range(3):
                dc = (kw - 1) * d
                t = kh * 3 + kw
                s = dr * W + dc
                if s == 0:
                    tap = cb
                else:
                    shifted = pltpu.roll(ci, shift=(-s) % HW, axis=1)
                    tap = pltpu.bitcast(shifted & maski(dr, dc), bf16)
                col_ref[t * C:(t + 1) * C, :] = tap

    def conv(col_ref, w_ref):
        y = jnp.dot(w_ref[...], col_ref[...], preferred_element_type=f32)
        return jnp.maximum(y, 0.0)

    G = x_ref.shape[0]
    cols = (col_a, col_b)
    n = 0
    for i in range(G):
        x = x_ref[i]
        build_col(cols[n % 2], x, 1)
        d1 = conv(cols[n % 2], w1_ref)
        n += 1
        o_ref[i] = x + d1
        build_col(cols[n % 2], d1, 2)
        d2 = conv(cols[n % 2], w2_ref)
        n += 1
        o_ref[i] += d2
        build_col(cols[n % 2], d2, 4)
        d3 = conv(cols[n % 2], w3_ref)
        n += 1
        o_ref[i] += d3


def _dblock(x_nchw, w1, w2, w3):
    B, C, H, W = x_nchw.shape
    HW = H * W
    x2 = x_nchw.reshape(B, C, HW)
    ws = [jnp.transpose(w.reshape(9 * C, C)).astype(jnp.bfloat16)
          for w in (w1, w2, w3)]

    flops = 3 * 2 * HW * (9 * C) * C * B
    bytes_accessed = 2 * B * C * HW * 4 + 3 * 9 * C * C * 2
    G = 16
    out = pl.pallas_call(
        functools.partial(_dblock_kernel, C=C, H=H, W=W),
        out_shape=jax.ShapeDtypeStruct((B, C, HW), x_nchw.dtype),
        grid=(B // G,),
        in_specs=[
            pl.BlockSpec((G, C, HW), lambda b: (b, 0, 0)),
            pl.BlockSpec(memory_space=pltpu.MemorySpace.VMEM),
            pl.BlockSpec(memory_space=pltpu.MemorySpace.VMEM),
            pl.BlockSpec(memory_space=pltpu.MemorySpace.VMEM),
        ],
        out_specs=pl.BlockSpec((G, C, HW), lambda b: (b, 0, 0)),
        scratch_shapes=[pltpu.VMEM((9 * C, HW), jnp.bfloat16),
                        pltpu.VMEM((9 * C, HW), jnp.bfloat16)],
        compiler_params=pltpu.CompilerParams(
            dimension_semantics=("parallel",)),
        cost_estimate=pl.CostEstimate(flops=flops, transcendentals=0,
                                      bytes_accessed=bytes_accessed),
    )(x2, *ws)
    return out.reshape(B, C, H, W)


def kernel(x_nchw, w1, w2, w3):
    return _dblock(x_nchw, w1, w2, w3)


# speedup vs baseline: 1.0061x; 1.0061x over previous
import functools

import jax
import jax.numpy as jnp
from jax import lax
from jax.experimental import pallas as pl
from jax.experimental.pallas import tpu as pltpu


def _dblock_kernel(x_ref, w1_ref, w2_ref, w3_ref, o_ref, col_a, col_b,
                   *, C, H, W):
    f32 = jnp.float32
    bf16 = jnp.bfloat16
    HW = H * W

    idx = lax.broadcasted_iota(jnp.int32, (1, HW), 1)
    row = idx // W
    col = idx - row * W

    def maski(dr, dc):
        m = None
        if dr < 0:
            m = row >= -dr
        elif dr > 0:
            m = row < H - dr
        if dc < 0:
            mc = col >= -dc
        elif dc > 0:
            mc = col < W - dc
        else:
            mc = None
        if m is None:
            m = mc
        elif mc is not None:
            m = m & mc
        return jnp.where(m, jnp.int32(-1), jnp.int32(0))

    def build_col(col_ref, cur, d):
        cb = cur.astype(bf16)
        ci = pltpu.bitcast(cb, jnp.int32)
        for kh in range(3):
            dr = (kh - 1) * d
            for kw in range(3):
                dc = (kw - 1) * d
                t = kh * 3 + kw
                s = dr * W + dc
                if s == 0:
                    tap = cb
                else:
                    shifted = pltpu.roll(ci, shift=(-s) % HW, axis=1)
                    tap = pltpu.bitcast(shifted & maski(dr, dc), bf16)
                col_ref[t * C:(t + 1) * C, :] = tap

    def conv(col_ref, w_ref):
        y = jnp.dot(w_ref[...], col_ref[...], preferred_element_type=f32)
        return jnp.maximum(y, 0.0)

    G = x_ref.shape[0]
    cols = (col_a, col_b)
    n = 0
    for i in range(G):
        x = x_ref[i]
        build_col(cols[n % 2], x, 1)
        d1 = conv(cols[n % 2], w1_ref)
        n += 1
        o_ref[i] = x + d1
        build_col(cols[n % 2], d1, 2)
        d2 = conv(cols[n % 2], w2_ref)
        n += 1
        o_ref[i] += d2
        build_col(cols[n % 2], d2, 4)
        d3 = conv(cols[n % 2], w3_ref)
        n += 1
        o_ref[i] += d3


def _dblock(x_nchw, w1, w2, w3):
    B, C, H, W = x_nchw.shape
    HW = H * W
    x2 = x_nchw.reshape(B, C, HW)
    ws = [jnp.transpose(w.reshape(9 * C, C)).astype(jnp.bfloat16)
          for w in (w1, w2, w3)]

    flops = 3 * 2 * HW * (9 * C) * C * B
    bytes_accessed = 2 * B * C * HW * 4 + 3 * 9 * C * C * 2
    G = 8
    out = pl.pallas_call(
        functools.partial(_dblock_kernel, C=C, H=H, W=W),
        out_shape=jax.ShapeDtypeStruct((B, C, HW), x_nchw.dtype),
        grid=(B // G,),
        in_specs=[
            pl.BlockSpec((G, C, HW), lambda b: (b, 0, 0)),
            pl.BlockSpec(memory_space=pltpu.MemorySpace.VMEM),
            pl.BlockSpec(memory_space=pltpu.MemorySpace.VMEM),
            pl.BlockSpec(memory_space=pltpu.MemorySpace.VMEM),
        ],
        out_specs=pl.BlockSpec((G, C, HW), lambda b: (b, 0, 0)),
        scratch_shapes=[pltpu.VMEM((9 * C, HW), jnp.bfloat16),
                        pltpu.VMEM((9 * C, HW), jnp.bfloat16)],
        compiler_params=pltpu.CompilerParams(
            dimension_semantics=("parallel",)),
        cost_estimate=pl.CostEstimate(flops=flops, transcendentals=0,
                                      bytes_accessed=bytes_accessed),
    )(x2, *ws)
    return out.reshape(B, C, H, W)


def kernel(x_nchw, w1, w2, w3):
    return _dblock(x_nchw, w1, w2, w3)


# speedup vs baseline: 1.0742x; 1.0677x over previous
import functools

import jax
import jax.numpy as jnp
from jax import lax
from jax.experimental import pallas as pl
from jax.experimental.pallas import tpu as pltpu


def _dblock_kernel(x_ref, w1_ref, w2_ref, w3_ref, o_ref, col_a, col_b,
                   *, C, H, W):
    f32 = jnp.float32
    bf16 = jnp.bfloat16
    HW = H * W

    idx = lax.broadcasted_iota(jnp.int32, (1, HW), 1)
    row = idx // W
    col = idx - row * W

    def maski(dr, dc):
        m = None
        if dr < 0:
            m = row >= -dr
        elif dr > 0:
            m = row < H - dr
        if dc < 0:
            mc = col >= -dc
        elif dc > 0:
            mc = col < W - dc
        else:
            mc = None
        if m is None:
            m = mc
        elif mc is not None:
            m = m & mc
        return jnp.where(m, jnp.int32(-1), jnp.int32(0))

    def build_col(col_ref, cb, ci, d):
        t = 0
        for kh in range(3):
            dr = (kh - 1) * d
            for kw in range(3):
                dc = (kw - 1) * d
                s = dr * W + dc
                if s == 0:
                    continue
                shifted = pltpu.roll(ci, shift=(-s) % HW, axis=1)
                col_ref[t * C:(t + 1) * C, :] = (
                    pltpu.bitcast(shifted & maski(dr, dc), bf16))
                t += 1

    def conv(col_ref, cur, d, w_ref):
        cb = cur.astype(bf16)
        ci = pltpu.bitcast(cb, jnp.int32)
        build_col(col_ref, cb, ci, d)
        y = (jnp.dot(w_ref[:, :8 * C], col_ref[...],
                     preferred_element_type=f32)
             + jnp.dot(w_ref[:, 8 * C:], cb, preferred_element_type=f32))
        return jnp.maximum(y, 0.0)

    G = x_ref.shape[0]
    cols = (col_a, col_b)
    n = 0
    for i in range(G):
        x = x_ref[i]
        d1 = conv(cols[n % 2], x, 1, w1_ref)
        n += 1
        o_ref[i] = x + d1
        d2 = conv(cols[n % 2], d1, 2, w2_ref)
        n += 1
        o_ref[i] += d2
        d3 = conv(cols[n % 2], d2, 4, w3_ref)
        n += 1
        o_ref[i] += d3


def _dblock(x_nchw, w1, w2, w3):
    B, C, H, W = x_nchw.shape
    HW = H * W
    x2 = x_nchw.reshape(B, C, HW)
    perm = jnp.array([0, 1, 2, 3, 5, 6, 7, 8, 4])
    ws = [jnp.transpose(w.reshape(9, C, C)[perm].reshape(9 * C, C))
          .astype(jnp.bfloat16) for w in (w1, w2, w3)]

    flops = 3 * 2 * HW * (9 * C) * C * B
    bytes_accessed = 2 * B * C * HW * 4 + 3 * 9 * C * C * 2
    G = next(g for g in (8, 4, 2, 1) if B % g == 0)
    out = pl.pallas_call(
        functools.partial(_dblock_kernel, C=C, H=H, W=W),
        out_shape=jax.ShapeDtypeStruct((B, C, HW), x_nchw.dtype),
        grid=(B // G,),
        in_specs=[
            pl.BlockSpec((G, C, HW), lambda b: (b, 0, 0)),
            pl.BlockSpec(memory_space=pltpu.MemorySpace.VMEM),
            pl.BlockSpec(memory_space=pltpu.MemorySpace.VMEM),
            pl.BlockSpec(memory_space=pltpu.MemorySpace.VMEM),
        ],
        out_specs=pl.BlockSpec((G, C, HW), lambda b: (b, 0, 0)),
        scratch_shapes=[pltpu.VMEM((8 * C, HW), jnp.bfloat16),
                        pltpu.VMEM((8 * C, HW), jnp.bfloat16)],
        compiler_params=pltpu.CompilerParams(
            dimension_semantics=("parallel",)),
        cost_estimate=pl.CostEstimate(flops=flops, transcendentals=0,
                                      bytes_accessed=bytes_accessed),
    )(x2, *ws)
    return out.reshape(B, C, H, W)


def kernel(x_nchw, w1, w2, w3):
    return _dblock(x_nchw, w1, w2, w3)
